# Initial kernel scaffold; baseline (speedup 1.0000x reference)
import jax
import jax.numpy as jnp
from jax.experimental import pallas as pl


def kernel(x, edge_index, batch_index, W_in, b_in, W1, b1, W_out, b_out):
    raise NotImplementedError("write your pallas kernel here")


# speedup vs baseline: 74.6620x; 74.6620x over previous
import functools

import jax
import jax.numpy as jnp
from jax import lax
from jax.experimental import pallas as pl
from jax.experimental.pallas import tpu as pltpu
from jax.experimental.pallas import tpu_sc as plsc

N = 10000
E = 320000
F_IN = 128
H = 7
C = 10
G = 64

NC = 2
NS = 16
L = 16
HP = 8
HH = HP // NC

EPT_DEG = E // (NC * NS)
EPT_MSG = E // NS


def _sc_mesh():
    return plsc.VectorSubcoreMesh(core_axis_name="c", subcore_axis_name="s")


_SC_PARAMS = pltpu.CompilerParams(needs_layout_passes=False)


def _deg_body(dst_hbm, zeros_hbm, out_hbm, dst_v, acc_v):
    c = lax.axis_index("c")
    s = lax.axis_index("s")
    wid = c * NS + s
    pltpu.sync_copy(zeros_hbm.at[0], acc_v)
    pltpu.sync_copy(dst_hbm.at[pl.ds(wid * EPT_DEG, EPT_DEG)], dst_v)
    ones = jnp.full((L,), 1.0, jnp.float32)

    def body(i, carry):
        dv = dst_v[pl.ds(i * L, L)]
        plsc.addupdate_scatter(acc_v, [dv], ones)
        return carry

    lax.fori_loop(0, EPT_DEG // L, body, 0)
    pltpu.sync_copy(acc_v, out_hbm.at[wid])


def _deg_parts(dst, zeros):
    return pl.kernel(
        _deg_body,
        out_type=jax.ShapeDtypeStruct((NC * NS, N), jnp.float32),
        mesh=_sc_mesh(),
        compiler_params=_SC_PARAMS,
        scratch_types=[
            pltpu.VMEM((EPT_DEG,), jnp.int32),
            pltpu.VMEM((N,), jnp.float32),
        ],
    )(dst, zeros)


def _proj_body(x_ref, wT_ref, parts_ref, hsT_ref, dinv_ref):
    deg = jnp.sum(parts_ref[...], axis=0, keepdims=True) + 1.0
    dinv = lax.rsqrt(deg)
    hT = lax.dot_general(
        wT_ref[...], x_ref[...], (((1,), (1,)), ((), ())),
        preferred_element_type=jnp.float32)
    hsT_ref[...] = hT * dinv
    dinv_ref[...] = dinv


def _proj(x, wT, parts):
    return pl.pallas_call(
        _proj_body,
        out_shape=[
            jax.ShapeDtypeStruct((HP, N), jnp.float32),
            jax.ShapeDtypeStruct((1, N), jnp.float32),
        ],
    )(x, wT, parts)


def _msg_body(hsT_hbm, src_hbm, dst_hbm, zeros_hbm, out_hbm,
              tab_v, acc_v, src_v, dst_v):
    c = lax.axis_index("c")
    s = lax.axis_index("s")
    pltpu.sync_copy(hsT_hbm.at[pl.ds(c * HH, HH)], tab_v)
    pltpu.sync_copy(zeros_hbm.at[pl.ds(0, HH)], acc_v)
    base = s * EPT_MSG
    pltpu.sync_copy(src_hbm.at[pl.ds(base, EPT_MSG)], src_v)
    pltpu.sync_copy(dst_hbm.at[pl.ds(base, EPT_MSG)], dst_v)
    jvs = [jnp.full((L,), j, jnp.int32) for j in range(HH)]

    def body(i, carry):
        sv = src_v[pl.ds(i * L, L)]
        dv = dst_v[pl.ds(i * L, L)]
        for j in range(HH):
            vals = plsc.load_gather(tab_v, [jvs[j], sv])
            plsc.addupdate_scatter(acc_v, [jvs[j], dv], vals)
        return carry

    lax.fori_loop(0, EPT_MSG // L, body, 0)
    pltpu.sync_copy(acc_v, out_hbm.at[c, s])


def _msg_parts(hsT, src, dst, zeros):
    return pl.kernel(
        _msg_body,
        out_type=jax.ShapeDtypeStruct((NC, NS, HH, N), jnp.float32),
        mesh=_sc_mesh(),
        compiler_params=_SC_PARAMS,
        scratch_types=[
            pltpu.VMEM((HH, N), jnp.float32),
            pltpu.VMEM((HH, N), jnp.float32),
            pltpu.VMEM((EPT_MSG,), jnp.int32),
            pltpu.VMEM((EPT_MSG,), jnp.int32),
        ],
    )(hsT, src, dst, zeros)


def _final_body(cparts_ref, hsT_ref, dinv_ref, bin_ref, bi_ref,
                woutT_ref, bout_ref, out_ref):
    halves = []
    for c in range(NC):
        acc = cparts_ref[c, 0]
        for s in range(1, NS):
            acc = acc + cparts_ref[c, s]
        halves.append(acc)
    ST = jnp.concatenate(halves, axis=0)
    outT = jnp.maximum(
        dinv_ref[...] * (ST + hsT_ref[...]) + bin_ref[...], 0.0)
    gids = lax.broadcasted_iota(jnp.int32, (G, N), 0)
    onehot = jnp.where(gids == bi_ref[...], 1.0, 0.0)
    pooledT = lax.dot_general(
        outT, onehot, (((1,), (1,)), ((), ())),
        preferred_element_type=jnp.float32)
    ones_row = jnp.ones((1, N), jnp.float32)
    counts = lax.dot_general(
        ones_row, onehot, (((1,), (1,)), ((), ())),
        preferred_element_type=jnp.float32)
    pooledT = pooledT / jnp.maximum(counts, 1.0)
    logitsT = jnp.dot(woutT_ref[...], pooledT,
                      preferred_element_type=jnp.float32) + bout_ref[...]
    m = jnp.max(logitsT, axis=0, keepdims=True)
    e = jnp.exp(logitsT - m)
    out_ref[...] = e / jnp.sum(e, axis=0, keepdims=True)


def _final(cparts, hsT, dinvT, bin_col, bi_row, woutT, bout_col):
    return pl.pallas_call(
        _final_body,
        out_shape=jax.ShapeDtypeStruct((C, G), jnp.float32),
    )(cparts, hsT, dinvT, bin_col, bi_row, woutT, bout_col)


def kernel(x, edge_index, batch_index, W_in, b_in, W1, b1, W_out, b_out):
    src = edge_index[0]
    dst = edge_index[1]
    zeros = jnp.zeros((HP, N), jnp.float32)

    wT = jnp.zeros((HP, F_IN), jnp.float32).at[:H].set(W_in.T)
    woutT = jnp.zeros((C, HP), jnp.float32).at[:, :H].set(W_out.T)
    bin_col = jnp.zeros((HP, 1), jnp.float32).at[:H, 0].set(b_in)
    bout_col = b_out.reshape(C, 1)
    bi_row = batch_index.reshape(1, N)

    deg_parts = _deg_parts(dst, zeros)
    hsT, dinvT = _proj(x, wT, deg_parts)
    cparts = _msg_parts(hsT, src, dst, zeros)
    probsT = _final(cparts, hsT, dinvT, bin_col, bi_row, woutT, bout_col)
    return probsT.T


# speedup vs baseline: 97.3532x; 1.3039x over previous
import functools

import jax
import jax.numpy as jnp
from jax import lax
from jax.experimental import pallas as pl
from jax.experimental.pallas import tpu as pltpu
from jax.experimental.pallas import tpu_sc as plsc

N = 10000
E = 320000
F_IN = 128
H = 7
C = 10
G = 64

NC = 2
NS = 16
L = 16
HP = 8
HH = HP // NC

EPT_DEG = E // (NC * NS)
EPT_MSG = E // NS


def _sc_mesh():
    return plsc.VectorSubcoreMesh(core_axis_name="c", subcore_axis_name="s")


_SC_PARAMS = pltpu.CompilerParams(needs_layout_passes=False)


def _deg_body(dst_hbm, zeros_hbm, out_hbm, dst_v, acc_v):
    c = lax.axis_index("c")
    s = lax.axis_index("s")
    wid = c * NS + s
    pltpu.sync_copy(zeros_hbm.at[0], acc_v)
    pltpu.sync_copy(dst_hbm.at[pl.ds(wid * EPT_DEG, EPT_DEG)], dst_v)
    ones = jnp.full((L,), 1.0, jnp.float32)
    U = 5

    def body(i, carry):
        dvs = [dst_v[pl.ds((i * U + u) * L, L)] for u in range(U)]
        for dv in dvs:
            plsc.addupdate_scatter(acc_v, [dv], ones)
        return carry

    lax.fori_loop(0, EPT_DEG // (L * U), body, 0)
    pltpu.sync_copy(acc_v, out_hbm.at[wid])


def _deg_parts(dst, zeros):
    return pl.kernel(
        _deg_body,
        out_type=jax.ShapeDtypeStruct((NC * NS, N), jnp.float32),
        mesh=_sc_mesh(),
        compiler_params=_SC_PARAMS,
        scratch_types=[
            pltpu.VMEM((EPT_DEG,), jnp.int32),
            pltpu.VMEM((N,), jnp.float32),
        ],
    )(dst, zeros)


def _proj_body(x_ref, wT_ref, parts_ref, hsT_ref, dinv_ref):
    deg = jnp.sum(parts_ref[...], axis=0, keepdims=True) + 1.0
    dinv = lax.rsqrt(deg)
    hT = lax.dot_general(
        wT_ref[...], x_ref[...], (((1,), (1,)), ((), ())),
        preferred_element_type=jnp.float32)
    hsT_ref[...] = hT * dinv
    dinv_ref[...] = dinv


def _proj(x, wT, parts):
    return pl.pallas_call(
        _proj_body,
        out_shape=[
            jax.ShapeDtypeStruct((HP, N), jnp.float32),
            jax.ShapeDtypeStruct((1, N), jnp.float32),
        ],
    )(x, wT, parts)


def _msg_body(hsT_hbm, src_hbm, dst_hbm, zeros_hbm, out_hbm,
              tab_v, acc_v, src_v, dst_v):
    c = lax.axis_index("c")
    s = lax.axis_index("s")
    pltpu.sync_copy(hsT_hbm.at[pl.ds(c * HH, HH)], tab_v)
    pltpu.sync_copy(zeros_hbm.at[pl.ds(0, HH)], acc_v)
    base = s * EPT_MSG
    pltpu.sync_copy(src_hbm.at[pl.ds(base, EPT_MSG)], src_v)
    pltpu.sync_copy(dst_hbm.at[pl.ds(base, EPT_MSG)], dst_v)
    jvs = [jnp.full((L,), j, jnp.int32) for j in range(HH)]
    U = 5

    def body(i, carry):
        svs = [src_v[pl.ds((i * U + u) * L, L)] for u in range(U)]
        dvs = [dst_v[pl.ds((i * U + u) * L, L)] for u in range(U)]
        vals = [[plsc.load_gather(tab_v, [jvs[j], svs[u]])
                 for j in range(HH)] for u in range(U)]
        for u in range(U):
            for j in range(HH):
                plsc.addupdate_scatter(acc_v, [jvs[j], dvs[u]], vals[u][j])
        return carry

    lax.fori_loop(0, EPT_MSG // (L * U), body, 0)
    pltpu.sync_copy(acc_v, out_hbm.at[c, s])


def _msg_parts(hsT, src, dst, zeros):
    return pl.kernel(
        _msg_body,
        out_type=jax.ShapeDtypeStruct((NC, NS, HH, N), jnp.float32),
        mesh=_sc_mesh(),
        compiler_params=_SC_PARAMS,
        scratch_types=[
            pltpu.VMEM((HH, N), jnp.float32),
            pltpu.VMEM((HH, N), jnp.float32),
            pltpu.VMEM((EPT_MSG,), jnp.int32),
            pltpu.VMEM((EPT_MSG,), jnp.int32),
        ],
    )(hsT, src, dst, zeros)


def _final_body(cparts_ref, hsT_ref, dinv_ref, bin_ref, bi_ref,
                woutT_ref, bout_ref, out_ref):
    halves = []
    for c in range(NC):
        acc = cparts_ref[c, 0]
        for s in range(1, NS):
            acc = acc + cparts_ref[c, s]
        halves.append(acc)
    ST = jnp.concatenate(halves, axis=0)
    outT = jnp.maximum(
        dinv_ref[...] * (ST + hsT_ref[...]) + bin_ref[...], 0.0)
    gids = lax.broadcasted_iota(jnp.int32, (G, N), 0)
    onehot = jnp.where(gids == bi_ref[...], 1.0, 0.0)
    pooledT = lax.dot_general(
        outT, onehot, (((1,), (1,)), ((), ())),
        preferred_element_type=jnp.float32)
    ones_row = jnp.ones((1, N), jnp.float32)
    counts = lax.dot_general(
        ones_row, onehot, (((1,), (1,)), ((), ())),
        preferred_element_type=jnp.float32)
    pooledT = pooledT / jnp.maximum(counts, 1.0)
    logitsT = jnp.dot(woutT_ref[...], pooledT,
                      preferred_element_type=jnp.float32) + bout_ref[...]
    m = jnp.max(logitsT, axis=0, keepdims=True)
    e = jnp.exp(logitsT - m)
    out_ref[...] = e / jnp.sum(e, axis=0, keepdims=True)


def _final(cparts, hsT, dinvT, bin_col, bi_row, woutT, bout_col):
    return pl.pallas_call(
        _final_body,
        out_shape=jax.ShapeDtypeStruct((C, G), jnp.float32),
    )(cparts, hsT, dinvT, bin_col, bi_row, woutT, bout_col)


def kernel(x, edge_index, batch_index, W_in, b_in, W1, b1, W_out, b_out):
    src = edge_index[0]
    dst = edge_index[1]
    zeros = jnp.zeros((HP, N), jnp.float32)

    wT = jnp.zeros((HP, F_IN), jnp.float32).at[:H].set(W_in.T)
    woutT = jnp.zeros((C, HP), jnp.float32).at[:, :H].set(W_out.T)
    bin_col = jnp.zeros((HP, 1), jnp.float32).at[:H, 0].set(b_in)
    bout_col = b_out.reshape(C, 1)
    bi_row = batch_index.reshape(1, N)

    deg_parts = _deg_parts(dst, zeros)
    hsT, dinvT = _proj(x, wT, deg_parts)
    cparts = _msg_parts(hsT, src, dst, zeros)
    probsT = _final(cparts, hsT, dinvT, bin_col, bi_row, woutT, bout_col)
    return probsT.T
